# baseline (speedup 1.0000x reference)
import functools

import jax
import jax.numpy as jnp
from jax import lax
from jax.experimental import pallas as pl
from jax.experimental.pallas import tpu as pltpu
from jax.experimental.pallas import tpu_sc as plsc

VOCAB = 100000
EMBED = 64
MAXPOS = 512
BATCH = 1024
SEQ = 512

NC, NS, L = 2, 16, 16
NW = NC * NS
CHUNK = 128
TOTAL = BATCH * SEQ
TOTC = TOTAL // CHUNK
CPW = TOTC // NW
CPS = SEQ // CHUNK
KV = EMBED // L
EPS = 1e-5
NG = 4
NO = 2
GAHEAD = 2
IAHEAD = 3
UNROLL = 8


def _rsqrt_scalar(x):
    i = lax.bitcast_convert_type(x, jnp.int32)
    i = jnp.int32(0x5F3759DF) - lax.shift_right_logical(i, 1)
    y = lax.bitcast_convert_type(i, jnp.float32)
    for _ in range(3):
        y = y * (jnp.float32(1.5) - jnp.float32(0.5) * x * y * y)
    return y


def _sc_body(text_hbm, wword_hbm, pos_hbm, par_hbm, out_hbm,
             pos_v, par_v, iring, gbuf, obuf, isem, gsem, osem):
    c = lax.axis_index("c")
    s = lax.axis_index("s")
    wid = s * NC + c
    base = wid * CPW

    pltpu.sync_copy(pos_hbm, pos_v)
    pltpu.sync_copy(par_hbm, par_v)

    tt0 = [par_v[0, pl.ds(k * L, L)] for k in range(KV)]

    def fold(q, carry):
        for k in range(KV):
            for h in (0, EMBED):
                sl = pl.ds(h + k * L, L)
                pos_v[q, sl] = pos_v[q, sl] + tt0[k]
        return carry

    lax.fori_loop(0, MAXPOS // 2, fold, 0, unroll=8)

    gam = [par_v[1, pl.ds(k * L, L)] for k in range(KV)]
    bet = [par_v[1, pl.ds(EMBED + k * L, L)] for k in range(KV)]

    def start_ids(ci, b):
        pltpu.async_copy(text_hbm.at[base + ci], iring.at[b], isem[b])

    def start_gather(ci, b):
        pltpu.make_async_copy(text_hbm.at[base + ci], iring.at[b],
                              isem[b]).wait()
        pltpu.async_copy(wword_hbm.at[iring.at[b]], gbuf.at[b], gsem[b])

    def compute(ci, g, o):
        rows = gbuf.at[g]
        dst = obuf.at[o]
        pbase2 = lax.rem(ci, CPS) * (CHUNK // 2)

        def grp(i, carry):
            t0 = i * UNROLL
            r0 = i * (UNROLL // 2)
            for u in range(UNROLL):
                t = t0 + u
                pr = pbase2 + r0 + (u // 2)
                hc = (u & 1) * EMBED
                x = [rows[t, pl.ds(k * L, L)] + pos_v[pr, pl.ds(hc + k * L, L)]
                     for k in range(KV)]
                sv = (x[0] + x[1]) + (x[2] + x[3])
                qv = (x[0] * x[0] + x[1] * x[1]) + (x[2] * x[2] + x[3] * x[3])
                mean = jnp.sum(sv) * jnp.float32(1.0 / EMBED)
                ex2 = jnp.sum(qv) * jnp.float32(1.0 / EMBED)
                rstd = _rsqrt_scalar(ex2 - mean * mean + jnp.float32(EPS))
                for k in range(KV):
                    dst[r0 + (u // 2), pl.ds(hc + k * L, L)] = \
                        (x[k] - mean) * rstd * gam[k] + bet[k]
            return carry

        lax.fori_loop(0, CHUNK // UNROLL, grp, 0, unroll=False)

    for j in range(IAHEAD):
        start_ids(j, j)
    for j in range(GAHEAD):
        start_gather(j, j)

    def step(ci, g, o):
        @pl.when(ci + IAHEAD < CPW)
        def _():
            start_ids(ci + IAHEAD, (g + IAHEAD) % NG)

        @pl.when(ci + GAHEAD < CPW)
        def _():
            start_gather(ci + GAHEAD, (g + GAHEAD) % NG)

        pltpu.make_async_copy(wword_hbm.at[iring.at[g]], gbuf.at[g],
                              gsem[g]).wait()

        @pl.when(ci >= NO)
        def _():
            orow = (base + ci - NO) * (CHUNK // 2)
            pltpu.make_async_copy(obuf.at[o], out_hbm.at[pl.ds(orow, CHUNK // 2)],
                                  osem[o]).wait()

        compute(ci, g, o)
        orow = (base + ci) * (CHUNK // 2)
        pltpu.async_copy(obuf.at[o], out_hbm.at[pl.ds(orow, CHUNK // 2)],
                         osem[o])

    def outer(i, carry):
        for j in range(NG):
            ci = i * NG + j
            step(ci, j, j % NO)
        return carry

    lax.fori_loop(0, CPW // NG, outer, 0, unroll=False)

    for j in range(NO):
        ci = CPW - NO + j
        orow = (base + ci) * (CHUNK // 2)
        pltpu.make_async_copy(obuf.at[ci % NO], out_hbm.at[pl.ds(orow, CHUNK // 2)],
                              osem[ci % NO]).wait()


@functools.partial(jax.jit, static_argnames=())
def _run(text2d, wword2, pos2, par2):
    mesh = plsc.VectorSubcoreMesh(core_axis_name="c", subcore_axis_name="s",
                                  num_cores=NC, num_subcores=NS)
    f = pl.kernel(
        _sc_body,
        out_type=jax.ShapeDtypeStruct((TOTAL // 2, 128), jnp.float32),
        mesh=mesh,
        compiler_params=pltpu.CompilerParams(needs_layout_passes=False,
                                             use_tc_tiling_on_sc=True),
        scratch_types=[
            pltpu.VMEM((MAXPOS // 2, 128), jnp.float32),
            pltpu.VMEM((8, 128), jnp.float32),
            pltpu.VMEM((NG, CHUNK), jnp.int32),
            pltpu.VMEM((NG, CHUNK, 128), jnp.float32),
            pltpu.VMEM((NO, CHUNK // 2, 128), jnp.float32),
            [pltpu.SemaphoreType.DMA] * NG,
            [pltpu.SemaphoreType.DMA] * NG,
            [pltpu.SemaphoreType.DMA] * NO,
        ],
    )
    return f(text2d, wword2, pos2, par2)


def kernel(text, W_word, W_pos, W_tt, ln_gamma, ln_beta):
    text2d = text.reshape(TOTC, CHUNK).astype(jnp.int32)
    wword2 = jnp.concatenate([W_word, W_word], axis=1)
    pos2 = W_pos.reshape(MAXPOS // 2, 128)
    par2 = jnp.concatenate([
        W_tt.reshape(1, 128),
        jnp.concatenate([ln_gamma, ln_beta]).reshape(1, 128),
        jnp.zeros((6, 128), jnp.float32),
    ])
    out = _run(text2d, wword2, pos2, par2)
    return out.reshape(BATCH, SEQ, EMBED)


# speedup vs baseline: 2.1295x; 2.1295x over previous
import functools

import jax
import jax.numpy as jnp
from jax import lax
from jax.experimental import pallas as pl
from jax.experimental.pallas import tpu as pltpu
from jax.experimental.pallas import tpu_sc as plsc

VOCAB = 100000
EMBED = 64
MAXPOS = 512
BATCH = 1024
SEQ = 512

NC, NS, L = 2, 16, 16
NW = NC * NS
CHUNK = 128
TOTAL = BATCH * SEQ
TOTC = TOTAL // CHUNK
CPW = TOTC // NW
CPS = SEQ // CHUNK
KV = EMBED // L
EPS = 1e-5
NG = 4
NO = 2
AHEAD = 2


def _rsqrt(x):
    i = lax.bitcast_convert_type(x, jnp.int32)
    i = jnp.int32(0x5F3759DF) - lax.shift_right_logical(i, 1)
    y = lax.bitcast_convert_type(i, jnp.float32)
    for _ in range(3):
        y = y * (jnp.float32(1.5) - jnp.float32(0.5) * x * y * y)
    return y


def _sc_body(text_hbm, wword_hbm, pos_hbm, tt_hbm, gam_hbm, beta_hbm, out_hbm,
             pos_v, par_v, idx_v, gbuf, obuf, gsem, osem):
    c = lax.axis_index("c")
    s = lax.axis_index("s")
    wid = s * NC + c
    base = wid * CPW

    pltpu.sync_copy(pos_hbm, pos_v)
    pltpu.sync_copy(tt_hbm, par_v.at[pl.ds(0, 2)])
    pltpu.sync_copy(gam_hbm, par_v.at[2])
    pltpu.sync_copy(beta_hbm, par_v.at[3])
    pltpu.sync_copy(text_hbm.at[pl.ds(base, CPW)], idx_v)

    tt0 = [par_v[0, pl.ds(k * L, L)] for k in range(KV)]

    @plsc.parallel_loop(0, MAXPOS, unroll=8)
    def _(p):
        for k in range(KV):
            sl = pl.ds(k * L, L)
            pos_v[p, sl] = pos_v[p, sl] + tt0[k]

    gam = [par_v[2, pl.ds(k * L, L)] for k in range(KV)]
    bet = [par_v[3, pl.ds(k * L, L)] for k in range(KV)]

    def start_gather(ci, g):
        pltpu.async_copy(wword_hbm.at[idx_v.at[ci]], gbuf.at[g], gsem[g])

    def compute(ci, g, o):
        rows = gbuf.at[g]
        dst = obuf.at[o]
        pbase = lax.rem(ci, CPS) * CHUNK

        @plsc.parallel_loop(0, CHUNK, unroll=8)
        def _(t):
            p = pbase + t
            x = [rows[t, pl.ds(k * L, L)] + pos_v[p, pl.ds(k * L, L)]
                 for k in range(KV)]
            sv = (x[0] + x[1]) + (x[2] + x[3])
            qv = (x[0] * x[0] + x[1] * x[1]) + (x[2] * x[2] + x[3] * x[3])
            mean = jnp.sum(sv) * jnp.float32(1.0 / EMBED)
            ex2 = jnp.sum(qv) * jnp.float32(1.0 / EMBED)
            rstd = _rsqrt(ex2 - mean * mean + jnp.float32(EPS))
            for k in range(KV):
                dst[t, pl.ds(k * L, L)] = (x[k] - mean) * rstd * gam[k] + bet[k]

    for j in range(AHEAD):
        start_gather(j, j)

    def step(ci, g, o):
        @pl.when(ci + AHEAD < CPW)
        def _():
            start_gather(ci + AHEAD, (g + AHEAD) % NG)

        pltpu.make_async_copy(wword_hbm.at[idx_v.at[ci]], gbuf.at[g],
                              gsem[g]).wait()

        @pl.when(ci >= NO)
        def _():
            pltpu.make_async_copy(obuf.at[o], out_hbm.at[base + ci - NO],
                                  osem[o]).wait()

        compute(ci, g, o)
        pltpu.async_copy(obuf.at[o], out_hbm.at[base + ci], osem[o])

    def outer(i, carry):
        for j in range(NG):
            ci = i * NG + j
            step(ci, j, j % NO)
        return carry

    lax.fori_loop(0, CPW // NG, outer, 0, unroll=False)

    for j in range(NO):
        ci = CPW - NO + j
        pltpu.make_async_copy(obuf.at[ci % NO], out_hbm.at[base + ci],
                              osem[ci % NO]).wait()


@functools.partial(jax.jit, static_argnames=())
def _run(text2d, W_word, W_pos, W_tt, ln_gamma, ln_beta):
    mesh = plsc.VectorSubcoreMesh(core_axis_name="c", subcore_axis_name="s",
                                  num_cores=NC, num_subcores=NS)
    f = pl.kernel(
        _sc_body,
        out_type=jax.ShapeDtypeStruct((TOTC, CHUNK, EMBED), jnp.float32),
        mesh=mesh,
        compiler_params=pltpu.CompilerParams(needs_layout_passes=False,
                                             use_tc_tiling_on_sc=False),
        scratch_types=[
            pltpu.VMEM((MAXPOS, EMBED), jnp.float32),
            pltpu.VMEM((4, EMBED), jnp.float32),
            pltpu.VMEM((CPW, CHUNK), jnp.int32),
            pltpu.VMEM((NG, CHUNK, EMBED), jnp.float32),
            pltpu.VMEM((NO, CHUNK, EMBED), jnp.float32),
            [pltpu.SemaphoreType.DMA] * NG,
            [pltpu.SemaphoreType.DMA] * NO,
        ],
    )
    return f(text2d, W_word, W_pos, W_tt, ln_gamma, ln_beta)


def kernel(text, W_word, W_pos, W_tt, ln_gamma, ln_beta):
    text2d = text.reshape(TOTC, CHUNK).astype(jnp.int32)
    out = _run(text2d, W_word, W_pos, W_tt, ln_gamma, ln_beta)
    return out.reshape(BATCH, SEQ, EMBED)


# speedup vs baseline: 2.1333x; 1.0018x over previous
import functools

import jax
import jax.numpy as jnp
from jax import lax
from jax.experimental import pallas as pl
from jax.experimental.pallas import tpu as pltpu
from jax.experimental.pallas import tpu_sc as plsc

VOCAB = 100000
EMBED = 64
MAXPOS = 512
BATCH = 1024
SEQ = 512

NC, NS, L = 2, 16, 16
NW = NC * NS
CHUNK = 128
TOTAL = BATCH * SEQ
TOTC = TOTAL // CHUNK
CPW = TOTC // NW
CPS = SEQ // CHUNK
KV = EMBED // L
EPS = 1e-5
NG = 4
NO = 2
AHEAD = 2


def _rsqrt(x):
    i = lax.bitcast_convert_type(x, jnp.int32)
    i = jnp.int32(0x5F3759DF) - lax.shift_right_logical(i, 1)
    y = lax.bitcast_convert_type(i, jnp.float32)
    for _ in range(3):
        y = y * (jnp.float32(1.5) - jnp.float32(0.5) * x * y * y)
    return y


def _sc_body(text_hbm, wword_hbm, pos_hbm, tt_hbm, gam_hbm, beta_hbm, out_hbm,
             pos_v, par_v, idx_v, gbuf, obuf, gsem, osem):
    c = lax.axis_index("c")
    s = lax.axis_index("s")
    wid = s * NC + c
    base = wid * CPW

    pltpu.sync_copy(pos_hbm, pos_v)
    pltpu.sync_copy(tt_hbm, par_v.at[pl.ds(0, 2)])
    pltpu.sync_copy(gam_hbm, par_v.at[2])
    pltpu.sync_copy(beta_hbm, par_v.at[3])
    pltpu.sync_copy(text_hbm.at[pl.ds(base, CPW)], idx_v)

    tt0 = [par_v[0, pl.ds(k * L, L)] for k in range(KV)]

    @plsc.parallel_loop(0, MAXPOS, unroll=8)
    def _(p):
        for k in range(KV):
            sl = pl.ds(k * L, L)
            pos_v[p, sl] = pos_v[p, sl] + tt0[k]

    gam = [par_v[2, pl.ds(k * L, L)] for k in range(KV)]
    bet = [par_v[3, pl.ds(k * L, L)] for k in range(KV)]

    def start_gather(ci, g):
        pltpu.async_copy(wword_hbm.at[idx_v.at[ci]], gbuf.at[g], gsem[g])

    def compute(ci, g, o):
        rows = gbuf.at[g]
        dst = obuf.at[o]
        pbase = lax.rem(ci, CPS) * CHUNK

        @plsc.parallel_loop(0, CHUNK, unroll=8)
        def _(t):
            p = pbase + t
            x = [rows[t, pl.ds(k * L, L)] + pos_v[p, pl.ds(k * L, L)]
                 for k in range(KV)]
            sv = (x[0] + x[1]) + (x[2] + x[3])
            qv = (x[0] * x[0] + x[1] * x[1]) + (x[2] * x[2] + x[3] * x[3])
            mean = jnp.sum(sv) * jnp.float32(1.0 / EMBED)
            ex2 = jnp.sum(qv) * jnp.float32(1.0 / EMBED)
            rstd = _rsqrt(ex2 - mean * mean + jnp.float32(EPS))
            for k in range(KV):
                dst[t, pl.ds(k * L, L)] = (x[k] - mean) * rstd * gam[k] + bet[k]

    for j in range(AHEAD):
        start_gather(j, j)

    def out_slice(r):
        b = lax.shift_right_logical(r, 2)
        s0 = lax.bitwise_and(r, CPS - 1) * CHUNK
        return out_hbm.at[b, pl.ds(s0, CHUNK)]

    def step(ci, g, o):
        @pl.when(ci + AHEAD < CPW)
        def _():
            start_gather(ci + AHEAD, (g + AHEAD) % NG)

        pltpu.make_async_copy(wword_hbm.at[idx_v.at[ci]], gbuf.at[g],
                              gsem[g]).wait()

        @pl.when(ci >= NO)
        def _():
            pltpu.make_async_copy(obuf.at[o], out_slice(base + ci - NO),
                                  osem[o]).wait()

        compute(ci, g, o)
        pltpu.async_copy(obuf.at[o], out_slice(base + ci), osem[o])

    def outer(i, carry):
        for j in range(NG):
            ci = i * NG + j
            step(ci, j, j % NO)
        return carry

    lax.fori_loop(0, CPW // NG, outer, 0, unroll=False)

    for j in range(NO):
        ci = CPW - NO + j
        pltpu.make_async_copy(obuf.at[ci % NO], out_slice(base + ci),
                              osem[ci % NO]).wait()


@functools.partial(jax.jit, static_argnames=())
def _run(text2d, W_word, W_pos, W_tt, ln_gamma, ln_beta):
    mesh = plsc.VectorSubcoreMesh(core_axis_name="c", subcore_axis_name="s",
                                  num_cores=NC, num_subcores=NS)
    f = pl.kernel(
        _sc_body,
        out_type=jax.ShapeDtypeStruct((BATCH, SEQ, EMBED), jnp.float32),
        mesh=mesh,
        compiler_params=pltpu.CompilerParams(needs_layout_passes=False,
                                             use_tc_tiling_on_sc=False),
        scratch_types=[
            pltpu.VMEM((MAXPOS, EMBED), jnp.float32),
            pltpu.VMEM((4, EMBED), jnp.float32),
            pltpu.VMEM((CPW, CHUNK), jnp.int32),
            pltpu.VMEM((NG, CHUNK, EMBED), jnp.float32),
            pltpu.VMEM((NO, CHUNK, EMBED), jnp.float32),
            [pltpu.SemaphoreType.DMA] * NG,
            [pltpu.SemaphoreType.DMA] * NO,
        ],
    )
    return f(text2d, W_word, W_pos, W_tt, ln_gamma, ln_beta)


def kernel(text, W_word, W_pos, W_tt, ln_gamma, ln_beta):
    text2d = text.reshape(TOTC, CHUNK).astype(jnp.int32)
    return _run(text2d, W_word, W_pos, W_tt, ln_gamma, ln_beta)
